# baseline (speedup 1.0000x reference)
import dataclasses
import functools

import numpy as np
import jax
import jax.numpy as jnp
from jax import lax
from jax.experimental import pallas as pl
from jax.experimental.pallas import tpu as pltpu
from jax.experimental.pallas import tpu_sc as plsc

_NUM_FIELDS = 26
_EMBED_DIM = 16
_EMBED_OUT = _NUM_FIELDS * _EMBED_DIM
_B = 16384
_N_IDX = _B * _NUM_FIELDS
_VOCAB = 100000 * _NUM_FIELDS
_OFFS = np.arange(_NUM_FIELDS, dtype=np.int32) * 100000

_NC = 2
_NS = 16
_NW = _NC * _NS
_PER_W = _N_IDX // _NW
_CHUNK = 3328
_N_CHUNKS = _PER_W // _CHUNK

_S_MAT = np.tile(np.eye(_EMBED_DIM, dtype=np.float32), (_NUM_FIELDS, 1))


def _sc_compiler_params():
    cp = pltpu.CompilerParams(use_tc_tiling_on_sc=False)
    if "needs_layout_passes" in pltpu.CompilerParams.__dataclass_fields__:
        cp = dataclasses.replace(cp, needs_layout_passes=False)
    return cp


@functools.lru_cache(maxsize=1)
def _build_sc_gather():
    @functools.partial(
        pl.kernel,
        out_type=[
            jax.ShapeDtypeStruct((_N_IDX, _EMBED_DIM), jnp.float32),
            jax.ShapeDtypeStruct((_N_IDX,), jnp.float32),
        ],
        mesh=plsc.VectorSubcoreMesh(core_axis_name="c", subcore_axis_name="s"),
        scratch_types=[
            pltpu.VMEM((_CHUNK,), jnp.int32),
            pltpu.VMEM((_CHUNK,), jnp.int32),
            pltpu.VMEM((_CHUNK,), jnp.int32),
            pltpu.VMEM((_CHUNK, _EMBED_DIM), jnp.float32),
            pltpu.VMEM((_CHUNK, _EMBED_DIM), jnp.float32),
            pltpu.VMEM((_CHUNK,), jnp.float32),
            pltpu.SemaphoreType.DMA,
            pltpu.SemaphoreType.DMA,
        ],
        compiler_params=_sc_compiler_params(),
    )
    def _sc_gather(eidx_hbm, lidx_hbm, lo_hbm, e_tab, l_tab,
                   emb_out, lval_out,
                   eidx_v, lidx_v, lo_v, erow_v, lrow_v, lval_v, sem_e, sem_l):
        wid = lax.axis_index("s") * _NC + lax.axis_index("c")
        base = wid * _PER_W

        @pl.loop(0, _PER_W, step=_CHUNK)
        def _(off):
            start = base + off
            pltpu.sync_copy(eidx_hbm.at[pl.ds(start, _CHUNK)], eidx_v)
            pltpu.sync_copy(lidx_hbm.at[pl.ds(start, _CHUNK)], lidx_v)
            pltpu.sync_copy(lo_hbm.at[pl.ds(start, _CHUNK)], lo_v)
            cp_e = pltpu.async_copy(e_tab.at[eidx_v], erow_v, sem_e)
            cp_l = pltpu.async_copy(l_tab.at[lidx_v], lrow_v, sem_l)
            cp_e.wait()
            cp_l.wait()

            @pl.loop(0, _CHUNK, step=16)
            def _(j):
                rows = lax.iota(jnp.int32, 16) + j
                cols = lo_v[pl.ds(j, 16)]
                lval_v[pl.ds(j, 16)] = plsc.load_gather(lrow_v, [rows, cols])

            pltpu.sync_copy(erow_v, emb_out.at[pl.ds(start, _CHUNK)])
            pltpu.sync_copy(lval_v, lval_out.at[pl.ds(start, _CHUNK)])

    return _sc_gather


_BLK = 2048
_NB = _B // _BLK


def _tc_body(emb_ref, lval_ref, W1_ref, b1_ref, g1_ref, be1_ref,
             W2_ref, b2_ref, g2_ref, be2_ref, W3_ref, sc_ref, S_ref,
             out_ref, h1_s, base_s):
    i = pl.program_id(0)
    M = emb_ref[...]
    h1 = jnp.dot(M, W1_ref[...], preferred_element_type=jnp.float32) + b1_ref[...]
    h1_s[pl.ds(i * _BLK, _BLK), :] = h1

    s = jnp.dot(M, S_ref[...], preferred_element_type=jnp.float32)
    fm = 0.5 * (jnp.sum(s * s, axis=1) - jnp.sum(M * M, axis=1))
    lin = jnp.sum(lval_ref[...], axis=1)
    base_s[pl.ds(i * _BLK, _BLK)] = lin + fm + sc_ref[0]

    @pl.when(i == _NB - 1)
    def _():
        H1 = h1_s[...]
        mu1 = jnp.mean(H1, axis=0, keepdims=True)
        d1 = H1 - mu1
        var1 = jnp.mean(d1 * d1, axis=0, keepdims=True)
        a1 = g1_ref[...] * lax.rsqrt(var1 + 1e-5)
        N1 = jnp.maximum(d1 * a1 + be1_ref[...], 0.0)
        H2 = jnp.dot(N1, W2_ref[...], preferred_element_type=jnp.float32) + b2_ref[...]
        mu2 = jnp.mean(H2, axis=0, keepdims=True)
        d2 = H2 - mu2
        var2 = jnp.mean(d2 * d2, axis=0, keepdims=True)
        a2 = g2_ref[...] * lax.rsqrt(var2 + 1e-5)
        N2 = jnp.maximum(d2 * a2 + be2_ref[...], 0.0)
        mlp = jnp.dot(N2, W3_ref[...], preferred_element_type=jnp.float32)[:, 0]
        z = base_s[...] + mlp
        e = jnp.exp(-jnp.abs(z))
        out_ref[...] = jnp.where(z >= 0, 1.0 / (1.0 + e), e / (1.0 + e))


def _tc_mlp(embM, lvalM, W1, b1, g1, be1, W2, b2, g2, be2, W3, sc):
    full = lambda shape: pl.BlockSpec(shape, lambda i: tuple(0 for _ in shape))
    return pl.pallas_call(
        _tc_body,
        grid=(_NB,),
        in_specs=[
            pl.BlockSpec((_BLK, _EMBED_OUT), lambda i: (i, 0)),
            pl.BlockSpec((_BLK, _NUM_FIELDS), lambda i: (i, 0)),
            full((_EMBED_OUT, 128)),
            full((1, 128)),
            full((1, 128)),
            full((1, 128)),
            full((128, 128)),
            full((1, 128)),
            full((1, 128)),
            full((1, 128)),
            full((128, 1)),
            pl.BlockSpec(memory_space=pltpu.SMEM),
            full((_EMBED_OUT, _EMBED_DIM)),
        ],
        out_specs=pl.BlockSpec((_B,), lambda i: (0,)),
        out_shape=jax.ShapeDtypeStruct((_B,), jnp.float32),
        scratch_shapes=[
            pltpu.VMEM((_B, 128), jnp.float32),
            pltpu.VMEM((_B,), jnp.float32),
        ],
    )(embM, lvalM, W1, b1.reshape(1, 128), g1.reshape(1, 128), be1.reshape(1, 128),
      W2, b2.reshape(1, 128), g2.reshape(1, 128), be2.reshape(1, 128),
      W3, sc, jnp.asarray(_S_MAT))


def kernel(x, E, L, bias, W1, b1, g1, be1, W2, b2, g2, be2, W3, b3):
    idx = x.astype(jnp.int32) + jnp.asarray(_OFFS)[None, :]
    eidx = idx.reshape(-1)
    lidx = eidx >> 4
    lo = eidx & 15
    L_r = L.reshape(_VOCAB // 16, 16)
    emb, lval = _build_sc_gather()(eidx, lidx, lo, E, L_r)
    embM = emb.reshape(_B, _EMBED_OUT)
    lvalM = lval.reshape(_B, _NUM_FIELDS)
    sc = (bias + b3).reshape(1)
    return _tc_mlp(embM, lvalM, W1, b1, g1, be1, W2, b2, g2, be2, W3, sc)
